# baseline (speedup 1.0000x reference)
import functools

import jax
import jax.numpy as jnp
from jax import lax
from jax.experimental import pallas as pl
from jax.experimental.pallas import tpu as pltpu
from jax.experimental.pallas import tpu_sc as plsc

B = 16384
K = 32
NC = 2
NS = 16
NW = NC * NS
BPW = B // NW
CHUNK = 128
NCHUNK = BPW // CHUNK
TILE = 128
R = 8
NROUND = BPW // R

_mesh = plsc.VectorSubcoreMesh(core_axis_name="c", subcore_axis_name="s")


@functools.partial(
    pl.kernel,
    mesh=_mesh,
    out_type=jax.ShapeDtypeStruct((B,), jnp.float32),
    compiler_params=pltpu.CompilerParams(needs_layout_passes=False),
    scratch_types=[
        pltpu.VMEM((2, NCHUNK, CHUNK), jnp.int32),
        pltpu.VMEM((R, K, TILE), jnp.float32),
        pltpu.VMEM((R, K, TILE), jnp.float32),
        pltpu.VMEM((16, 16), jnp.float32),
        pltpu.VMEM((BPW,), jnp.float32),
        pltpu.VMEM((48,), jnp.float32),
        pltpu.SemaphoreType.DMA,
    ],
)
def _gmf_sc(u_hbm, v_hbm, utt_hbm, vtt_hbm, wb_hbm, out_hbm,
            idx_uv, ublk, vblk, tbuf, outv, wbv, sem):
    wid = lax.axis_index("s") * NC + lax.axis_index("c")
    base = wid * BPW

    pltpu.sync_copy(u_hbm.at[pl.ds(wid * NCHUNK, NCHUNK)], idx_uv.at[0])
    pltpu.sync_copy(v_hbm.at[pl.ds(wid * NCHUNK, NCHUNK)], idx_uv.at[1])
    pltpu.sync_copy(wb_hbm, wbv)

    iota = lax.iota(jnp.int32, 16)
    w0 = wbv[pl.ds(0, 16)]
    w1 = wbv[pl.ds(16, 16)]
    bias = wbv[pl.ds(32, 16)]

    def round_body(rnd, carry):
        p = rnd * 16
        c = p // CHUNK
        off = p % CHUNK
        uvec = idx_uv[0, c, pl.ds(off, 16)]
        vvec = idx_uv[1, c, pl.ds(off, 16)]

        for half in range(2):
            copies = []
            for l in range(R):
                su = pl.multiple_of(uvec[half * R + l] & -TILE, TILE)
                sv = pl.multiple_of(vvec[half * R + l] & -TILE, TILE)
                copies.append(pltpu.async_copy(
                    utt_hbm.at[:, pl.ds(su, TILE)], ublk.at[l], sem))
                copies.append(pltpu.async_copy(
                    vtt_hbm.at[:, pl.ds(sv, TILE)], vblk.at[l], sem))
            for cp in copies:
                cp.wait()

            for l in range(R):
                cl = jnp.full((16,), l, jnp.int32)
                ou = jnp.full((16,), uvec[half * R + l] & (TILE - 1),
                              jnp.int32)
                ov = jnp.full((16,), vvec[half * R + l] & (TILE - 1),
                              jnp.int32)
                u0 = plsc.load_gather(ublk, [cl, iota, ou])
                u1 = plsc.load_gather(ublk, [cl, iota + 16, ou])
                v0 = plsc.load_gather(vblk, [cl, iota, ov])
                v1 = plsc.load_gather(vblk, [cl, iota + 16, ov])
                tbuf[half * R + l, :] = u0 * v0 * w0 + u1 * v1 * w1

        def j_body(j, acc):
            cj = jnp.full((16,), j, jnp.int32)
            return acc + plsc.load_gather(tbuf, [iota, cj])

        acc = lax.fori_loop(0, 16, j_body, jnp.zeros((16,), jnp.float32))
        x = acc + bias
        y = 1.0 / (1.0 + jnp.exp(-x))
        outv[pl.ds(p, 16)] = y
        return carry

    lax.fori_loop(0, BPW // 16, round_body, 0)

    pltpu.sync_copy(outv, out_hbm.at[pl.ds(base, BPW)])


def kernel(u, v, u_table, v_table, h_W, h_b):
    u2 = u.reshape(B // CHUNK, CHUNK)
    v2 = v.reshape(B // CHUNK, CHUNK)
    wb = jnp.concatenate(
        [h_W.reshape(K), jnp.broadcast_to(h_b.reshape(1), (16,))])
    out = _gmf_sc(u2, v2, u_table.T, v_table.T, wb)
    return out.reshape(B, 1)


# speedup vs baseline: 1.0400x; 1.0400x over previous
import functools

import jax
import jax.numpy as jnp
from jax import lax
from jax.experimental import pallas as pl
from jax.experimental.pallas import tpu as pltpu
from jax.experimental.pallas import tpu_sc as plsc

B = 16384
K = 32
NC = 2
NS = 16
NW = NC * NS
BPW = B // NW
CHUNK = 128
NCHUNK = BPW // CHUNK
TILE = 128
NG = BPW // 16

_mesh = plsc.VectorSubcoreMesh(core_axis_name="c", subcore_axis_name="s")


@functools.partial(
    pl.kernel,
    mesh=_mesh,
    out_type=jax.ShapeDtypeStruct((B,), jnp.float32),
    compiler_params=pltpu.CompilerParams(needs_layout_passes=False),
    scratch_types=[
        pltpu.VMEM((2, NCHUNK, CHUNK), jnp.int32),
        pltpu.VMEM((2, 16, 16, TILE), jnp.float32),
        pltpu.VMEM((4, 16, 16), jnp.float32),
        pltpu.VMEM((16, 16), jnp.float32),
        pltpu.VMEM((BPW,), jnp.float32),
        pltpu.VMEM((48,), jnp.float32),
        pltpu.SemaphoreType.DMA,
        pltpu.SemaphoreType.DMA,
    ],
)
def _gmf_sc(u_hbm, v_hbm, utt_hbm, vtt_hbm, wb_hbm, out_hbm,
            idx_uv, blk, cols, tbuf, outv, wbv, sem0, sem1):
    wid = lax.axis_index("s") * NC + lax.axis_index("c")
    base = wid * BPW

    pltpu.sync_copy(u_hbm.at[pl.ds(wid * NCHUNK, NCHUNK)], idx_uv.at[0])
    pltpu.sync_copy(v_hbm.at[pl.ds(wid * NCHUNK, NCHUNK)], idx_uv.at[1])
    pltpu.sync_copy(wb_hbm, wbv)

    iota = lax.iota(jnp.int32, 16)
    w0 = wbv[pl.ds(0, 16)]
    w1 = wbv[pl.ds(16, 16)]
    bias = wbv[pl.ds(32, 16)]
    sems = (sem0, sem1)
    tabs = (utt_hbm, vtt_hbm)

    def idx_vec(t, g):
        c = g // 8
        off = (g % 8) * 16
        return idx_uv[t, c, pl.ds(off, 16)]

    def fire(g, k):
        t, h, par = k // 2, k % 2, k % 2
        vec = idx_vec(t, g)
        for l in range(16):
            s = pl.multiple_of(vec[l] & -TILE, TILE)
            pltpu.async_copy(
                tabs[t].at[pl.ds(h * 16, 16), pl.ds(s, TILE)],
                blk.at[par, l], sems[par])

    fire(0, 0)

    def group_body(g, carry):
        for k in range(4):
            t, h, par = k // 2, k % 2, k % 2
            if k < 3:
                fire(g, k + 1)
            else:
                @pl.when(g != NG - 1)
                def _():
                    fire(g + 1, 0)
            for l in range(16):
                pltpu.make_async_copy(
                    tabs[t].at[pl.ds(0, 16), pl.ds(0, TILE)],
                    blk.at[par, l], sems[par]).wait()
            vec = idx_vec(t, g)
            for l in range(16):
                cl = jnp.full((16,), l, jnp.int32)
                o = jnp.full((16,), vec[l] & (TILE - 1), jnp.int32)
                cols[k, l, :] = plsc.load_gather(blk.at[par], [cl, iota, o])

        for l in range(16):
            u0 = cols[0, l, :]
            u1 = cols[1, l, :]
            v0 = cols[2, l, :]
            v1 = cols[3, l, :]
            tbuf[l, :] = u0 * v0 * w0 + u1 * v1 * w1

        def j_body(j, acc):
            cj = jnp.full((16,), j, jnp.int32)
            return acc + plsc.load_gather(tbuf, [iota, cj])

        acc = lax.fori_loop(0, 16, j_body, jnp.zeros((16,), jnp.float32))
        x = acc + bias
        y = 1.0 / (1.0 + jnp.exp(-x))
        outv[pl.ds(g * 16, 16)] = y
        return carry

    lax.fori_loop(0, NG, group_body, 0)

    pltpu.sync_copy(outv, out_hbm.at[pl.ds(base, BPW)])


def kernel(u, v, u_table, v_table, h_W, h_b):
    u2 = u.reshape(B // CHUNK, CHUNK)
    v2 = v.reshape(B // CHUNK, CHUNK)
    wb = jnp.concatenate(
        [h_W.reshape(K), jnp.broadcast_to(h_b.reshape(1), (16,))])
    out = _gmf_sc(u2, v2, u_table.T, v_table.T, wb)
    return out.reshape(B, 1)
